# baseline (speedup 1.0000x reference)
import functools

import jax
import jax.numpy as jnp
from jax import lax
from jax.experimental import pallas as pl
from jax.experimental.pallas import tpu as pltpu
from jax.experimental.pallas import tpu_sc as plsc

NC = 2
NS = 16
NW = NC * NS
LANES = 16


def _fori(n, body):
    lax.fori_loop(jnp.int32(0), jnp.int32(n), body, jnp.int32(0))


def _sc_agg_body(NP, D, E, C, R, LI, LD, x_hbm, eattr_hbm, src_hbm, dst_hbm,
                 zeros_hbm, out_hbm, *refs):
    si = refs[0:R]
    di = refs[R:2 * R]
    rows = refs[2 * R:3 * R]
    ein = refs[3 * R:4 * R]
    agg_sh = refs[4 * R]
    sxs = refs[4 * R + 1:4 * R + 1 + R]
    sxd = refs[4 * R + 1 + R:4 * R + 1 + 2 * R]
    sg = refs[4 * R + 1 + 2 * R:4 * R + 1 + 3 * R]
    se = refs[4 * R + 1 + 3 * R:4 * R + 1 + 4 * R]
    ss = refs[4 * R + 1 + 4 * R:4 * R + 1 + 5 * R]
    c = lax.axis_index("c")
    s = lax.axis_index("s")
    wid = c * jnp.int32(NS) + s
    epw = E // NW
    nchunk = epw // C
    rpt = NP // NS
    kd = D // LANES

    row0 = s * jnp.int32(rpt)
    pltpu.sync_copy(zeros_hbm, agg_sh.at[pl.ds(row0, rpt)])
    plsc.subcore_barrier()

    ebase = wid * jnp.int32(epw)

    def issue_idx(j, z):
        pltpu.async_copy(src_hbm.at[pl.ds(ebase + j * jnp.int32(C), C)],
                         si[z], sxs[z])
        pltpu.async_copy(dst_hbm.at[pl.ds(ebase + j * jnp.int32(C), C)],
                         di[z], sxd[z])

    def issue_data(j, z):
        pltpu.make_async_copy(src_hbm.at[pl.ds(0, C)], si[z], sxs[z]).wait()
        pltpu.async_copy(x_hbm.at[si[z]], rows[z], sg[z])
        pltpu.async_copy(eattr_hbm.at[pl.ds(ebase + j * jnp.int32(C), C)],
                         ein[z], se[z])

    def process(j, z):
        pltpu.make_async_copy(eattr_hbm.at[pl.ds(0, C)], ein[z],
                              se[z]).wait()
        pltpu.make_async_copy(x_hbm.at[si[z]], rows[z], sg[z]).wait()

        def relu_row(q, _):
            for t in range(4):
                r = q * jnp.int32(4) + jnp.int32(t)
                for k in range(kd):
                    sl = pl.ds(k * LANES, LANES)
                    rows[z][r, sl] = jnp.maximum(
                        rows[z][r, sl] + ein[z][r, sl], 0.0)
            return jnp.int32(0)
        _fori(C // 4, relu_row)

        pltpu.make_async_copy(dst_hbm.at[pl.ds(0, C)], di[z], sxd[z]).wait()
        dvec = di[z][...]
        pltpu.async_copy(rows[z], agg_sh.at[dvec], ss[z], add=True)

        @pl.when(j >= jnp.int32(R - LD))
        def _():
            pltpu.make_async_copy(rows[(z + LD) % R], agg_sh.at[dvec],
                                  ss[(z + LD) % R]).wait()

        @pl.when(j + jnp.int32(LI) < jnp.int32(nchunk))
        def _():
            issue_idx(j + jnp.int32(LI), (z + LI) % R)

        @pl.when(j + jnp.int32(LD) < jnp.int32(nchunk))
        def _():
            issue_data(j + jnp.int32(LD), (z + LD) % R)

    for q in range(LI):
        issue_idx(jnp.int32(q), q)
    for q in range(LD):
        issue_data(jnp.int32(q), q)

    def ring_loop(u, _):
        j0 = u * jnp.int32(R)
        for z in range(R):
            j = j0 + jnp.int32(z)

            @pl.when(j < jnp.int32(nchunk))
            def _(j=j, z=z):
                process(j, z)
        return jnp.int32(0)
    _fori((nchunk + R - 1) // R, ring_loop)

    dvec0 = di[0][...]
    for t in range(R - LD):
        pltpu.make_async_copy(rows[(nchunk - (R - LD) + t) % R],
                              agg_sh.at[dvec0],
                              ss[(nchunk - (R - LD) + t) % R]).wait()

    plsc.subcore_barrier()
    pltpu.sync_copy(agg_sh.at[pl.ds(row0, rpt)],
                    out_hbm.at[c, pl.ds(row0, rpt)])


def _sc_agg(x, eattr, src, dst, C=16, R=6, LI=6, LD=5):
    N, D = x.shape
    E = src.shape[0]
    rpt = (-(-N // NS) + 7) // 8 * 8
    NP = NS * rpt
    assert E % (NW * C) == 0 and D % LANES == 0
    mesh = plsc.VectorSubcoreMesh(core_axis_name="c", subcore_axis_name="s")
    f = pl.kernel(
        functools.partial(_sc_agg_body, NP, D, E, C, R, LI, LD),
        out_type=jax.ShapeDtypeStruct((NC, NP, D), jnp.float32),
        mesh=mesh,
        scratch_types=(
            [pltpu.VMEM((C,), jnp.int32)] * (2 * R)
            + [pltpu.VMEM((C, D), jnp.float32)] * (2 * R)
            + [pltpu.VMEM_SHARED((NP, D), jnp.float32)]
            + [pltpu.SemaphoreType.DMA] * (5 * R)
        ),
    )
    zeros = jnp.zeros((rpt, D), jnp.float32)
    return f(x, eattr, src, dst, zeros)


def _mlp0_body(x_ref, p_ref, w1_ref, b1_ref, w2_ref, b2_ref, o_ref):
    n = x_ref.shape[0]
    out = x_ref[...] + p_ref[0][:n] + p_ref[1][:n]
    h = jnp.maximum(
        jnp.dot(out, w1_ref[...], preferred_element_type=jnp.float32)
        + b1_ref[...], 0.0)
    h = jnp.dot(h, w2_ref[...], preferred_element_type=jnp.float32) + b2_ref[...]
    o_ref[...] = jnp.maximum(h, 0.0)


def _final_body(x0_ref, h_ref, p_ref, w1_ref, b1_ref, w2_ref, b2_ref,
                lng_ref, lnb_ref, gnw_ref, gnb_ref, gna_ref, o_ref):
    n = h_ref.shape[0]
    out = h_ref[...] + p_ref[0][:n] + p_ref[1][:n]
    h = jnp.maximum(
        jnp.dot(out, w1_ref[...], preferred_element_type=jnp.float32)
        + b1_ref[...], 0.0)
    h = jnp.dot(h, w2_ref[...], preferred_element_type=jnp.float32) + b2_ref[...]
    mu = jnp.mean(h, axis=-1, keepdims=True)
    var = jnp.mean((h - mu) ** 2, axis=-1, keepdims=True)
    h = (h - mu) * lax.rsqrt(var + 1e-5) * lng_ref[...] + lnb_ref[...]
    gmean = jnp.mean(h, axis=0, keepdims=True)
    sub = h - gna_ref[...] * gmean
    gvar = jnp.mean(sub * sub, axis=0, keepdims=True)
    h = gnw_ref[...] * sub * lax.rsqrt(gvar + 1e-5) + gnb_ref[...]
    o_ref[...] = jnp.maximum(h, 0.0) + x0_ref[...]


def kernel(node_hidden, edge_hidden, edge_index, W1_0, b1_0, W2_0, b2_0,
           W1_1, b1_1, W2_1, b2_1, ln_gamma, ln_beta, gn_weight, gn_bias,
           gn_alpha):
    N, D = node_hidden.shape
    ei = edge_index.astype(jnp.int32)
    src, dst = ei[0], ei[1]
    x = node_hidden.astype(jnp.float32)
    eattr = edge_hidden.astype(jnp.float32)
    r2 = lambda v: v.reshape(1, D).astype(jnp.float32)

    p0 = _sc_agg(x, eattr, src, dst)
    h1 = pl.pallas_call(
        _mlp0_body,
        out_shape=jax.ShapeDtypeStruct((N, D), jnp.float32),
    )(x, p0, W1_0, r2(b1_0), W2_0, r2(b2_0))

    p1 = _sc_agg(h1, eattr, src, dst)
    out = pl.pallas_call(
        _final_body,
        out_shape=jax.ShapeDtypeStruct((N, D), jnp.float32),
    )(x, h1, p1, W1_1, r2(b1_1), W2_1, r2(b2_1),
      r2(ln_gamma), r2(ln_beta), r2(gn_weight), r2(gn_bias), r2(gn_alpha))
    return out


# speedup vs baseline: 1.0035x; 1.0035x over previous
import functools

import jax
import jax.numpy as jnp
from jax import lax
from jax.experimental import pallas as pl
from jax.experimental.pallas import tpu as pltpu
from jax.experimental.pallas import tpu_sc as plsc

NC = 2
NS = 16
NW = NC * NS
LANES = 16


def _fori(n, body):
    lax.fori_loop(jnp.int32(0), jnp.int32(n), body, jnp.int32(0))


def _sc_agg_body(NP, D, E, C, R, LI, LD, x_hbm, eattr_hbm, src_hbm, dst_hbm,
                 zeros_hbm, out_hbm, *refs):
    si = refs[0:R]
    di = refs[R:2 * R]
    rows = refs[2 * R:3 * R]
    ein = refs[3 * R:4 * R]
    agg_sh = refs[4 * R]
    sxs = refs[4 * R + 1:4 * R + 1 + R]
    sxd = refs[4 * R + 1 + R:4 * R + 1 + 2 * R]
    sg = refs[4 * R + 1 + 2 * R:4 * R + 1 + 3 * R]
    se = refs[4 * R + 1 + 3 * R:4 * R + 1 + 4 * R]
    ss = refs[4 * R + 1 + 4 * R:4 * R + 1 + 5 * R]
    c = lax.axis_index("c")
    s = lax.axis_index("s")
    wid = c * jnp.int32(NS) + s
    epw = E // NW
    nchunk = epw // C
    rpt = NP // NS
    kd = D // LANES

    row0 = s * jnp.int32(rpt)
    pltpu.sync_copy(zeros_hbm, agg_sh.at[pl.ds(row0, rpt)])
    plsc.subcore_barrier()

    ebase = wid * jnp.int32(epw)

    def issue_idx(j, z):
        pltpu.async_copy(src_hbm.at[pl.ds(ebase + j * jnp.int32(C), C)],
                         si[z], sxs[z])
        pltpu.async_copy(dst_hbm.at[pl.ds(ebase + j * jnp.int32(C), C)],
                         di[z], sxd[z])

    def issue_data(j, z):
        pltpu.make_async_copy(src_hbm.at[pl.ds(0, C)], si[z], sxs[z]).wait()
        pltpu.async_copy(x_hbm.at[si[z]], rows[z], sg[z])
        pltpu.async_copy(eattr_hbm.at[pl.ds(ebase + j * jnp.int32(C), C)],
                         ein[z], se[z])

    def process(j, z):
        pltpu.make_async_copy(eattr_hbm.at[pl.ds(0, C)], ein[z],
                              se[z]).wait()
        pltpu.make_async_copy(x_hbm.at[si[z]], rows[z], sg[z]).wait()

        def relu_row(r, _):
            for k in range(kd):
                sl = pl.ds(k * LANES, LANES)
                rows[z][r, sl] = jnp.maximum(
                    rows[z][r, sl] + ein[z][r, sl], 0.0)
            return jnp.int32(0)
        _fori(C, relu_row)

        pltpu.make_async_copy(dst_hbm.at[pl.ds(0, C)], di[z], sxd[z]).wait()
        dvec = di[z][...]
        pltpu.async_copy(rows[z], agg_sh.at[dvec], ss[z], add=True)

        @pl.when(j >= jnp.int32(R - LD))
        def _():
            pltpu.make_async_copy(rows[(z + LD) % R], agg_sh.at[dvec],
                                  ss[(z + LD) % R]).wait()

        @pl.when(j + jnp.int32(LI) < jnp.int32(nchunk))
        def _():
            issue_idx(j + jnp.int32(LI), (z + LI) % R)

        @pl.when(j + jnp.int32(LD) < jnp.int32(nchunk))
        def _():
            issue_data(j + jnp.int32(LD), (z + LD) % R)

    for q in range(LI):
        issue_idx(jnp.int32(q), q)
    for q in range(LD):
        issue_data(jnp.int32(q), q)

    def ring_loop(u, _):
        j0 = u * jnp.int32(R)
        for z in range(R):
            j = j0 + jnp.int32(z)

            @pl.when(j < jnp.int32(nchunk))
            def _(j=j, z=z):
                process(j, z)
        return jnp.int32(0)
    _fori((nchunk + R - 1) // R, ring_loop)

    dvec0 = di[0][...]
    for t in range(R - LD):
        pltpu.make_async_copy(rows[(nchunk - (R - LD) + t) % R],
                              agg_sh.at[dvec0],
                              ss[(nchunk - (R - LD) + t) % R]).wait()

    plsc.subcore_barrier()
    pltpu.sync_copy(agg_sh.at[pl.ds(row0, rpt)],
                    out_hbm.at[c, pl.ds(row0, rpt)])


def _sc_agg(x, eattr, src, dst, C=16, R=6, LI=6, LD=5):
    N, D = x.shape
    E = src.shape[0]
    rpt = (-(-N // NS) + 7) // 8 * 8
    NP = NS * rpt
    assert E % (NW * C) == 0 and D % LANES == 0
    mesh = plsc.VectorSubcoreMesh(core_axis_name="c", subcore_axis_name="s")
    f = pl.kernel(
        functools.partial(_sc_agg_body, NP, D, E, C, R, LI, LD),
        out_type=jax.ShapeDtypeStruct((NC, NP, D), jnp.float32),
        mesh=mesh,
        scratch_types=(
            [pltpu.VMEM((C,), jnp.int32)] * (2 * R)
            + [pltpu.VMEM((C, D), jnp.float32)] * (2 * R)
            + [pltpu.VMEM_SHARED((NP, D), jnp.float32)]
            + [pltpu.SemaphoreType.DMA] * (5 * R)
        ),
    )
    zeros = jnp.zeros((rpt, D), jnp.float32)
    return f(x, eattr, src, dst, zeros)


def _mlp0_body(x_ref, p_ref, w1_ref, b1_ref, w2_ref, b2_ref, o_ref):
    n = x_ref.shape[0]
    out = x_ref[...] + p_ref[0][:n] + p_ref[1][:n]
    h = jnp.maximum(
        jnp.dot(out, w1_ref[...], preferred_element_type=jnp.float32)
        + b1_ref[...], 0.0)
    h = jnp.dot(h, w2_ref[...], preferred_element_type=jnp.float32) + b2_ref[...]
    o_ref[...] = jnp.maximum(h, 0.0)


def _final_body(x0_ref, h_ref, p_ref, w1_ref, b1_ref, w2_ref, b2_ref,
                lng_ref, lnb_ref, gnw_ref, gnb_ref, gna_ref, o_ref):
    n = h_ref.shape[0]
    out = h_ref[...] + p_ref[0][:n] + p_ref[1][:n]
    h = jnp.maximum(
        jnp.dot(out, w1_ref[...], preferred_element_type=jnp.float32)
        + b1_ref[...], 0.0)
    h = jnp.dot(h, w2_ref[...], preferred_element_type=jnp.float32) + b2_ref[...]
    mu = jnp.mean(h, axis=-1, keepdims=True)
    var = jnp.mean((h - mu) ** 2, axis=-1, keepdims=True)
    h = (h - mu) * lax.rsqrt(var + 1e-5) * lng_ref[...] + lnb_ref[...]
    gmean = jnp.mean(h, axis=0, keepdims=True)
    sub = h - gna_ref[...] * gmean
    gvar = jnp.mean(sub * sub, axis=0, keepdims=True)
    h = gnw_ref[...] * sub * lax.rsqrt(gvar + 1e-5) + gnb_ref[...]
    o_ref[...] = jnp.maximum(h, 0.0) + x0_ref[...]


def kernel(node_hidden, edge_hidden, edge_index, W1_0, b1_0, W2_0, b2_0,
           W1_1, b1_1, W2_1, b2_1, ln_gamma, ln_beta, gn_weight, gn_bias,
           gn_alpha):
    N, D = node_hidden.shape
    ei = edge_index.astype(jnp.int32)
    src, dst = ei[0], ei[1]
    x = node_hidden.astype(jnp.float32)
    eattr = edge_hidden.astype(jnp.float32)
    r2 = lambda v: v.reshape(1, D).astype(jnp.float32)

    p0 = _sc_agg(x, eattr, src, dst)
    h1 = pl.pallas_call(
        _mlp0_body,
        out_shape=jax.ShapeDtypeStruct((N, D), jnp.float32),
    )(x, p0, W1_0, r2(b1_0), W2_0, r2(b2_0))

    p1 = _sc_agg(h1, eattr, src, dst)
    out = pl.pallas_call(
        _final_body,
        out_shape=jax.ShapeDtypeStruct((N, D), jnp.float32),
    )(x, h1, p1, W1_1, r2(b1_1), W2_1, r2(b2_1),
      r2(ln_gamma), r2(ln_beta), r2(gn_weight), r2(gn_bias), r2(gn_alpha))
    return out


# speedup vs baseline: 1.0164x; 1.0129x over previous
import functools

import jax
import jax.numpy as jnp
from jax import lax
from jax.experimental import pallas as pl
from jax.experimental.pallas import tpu as pltpu
from jax.experimental.pallas import tpu_sc as plsc

NC = 2
NS = 16
NW = NC * NS
LANES = 16


def _fori(n, body):
    lax.fori_loop(jnp.int32(0), jnp.int32(n), body, jnp.int32(0))


def _sc_agg_body(NP, D, E, C, R, LI, LD, x_hbm, eattr_hbm, src_hbm, dst_hbm,
                 out_hbm, *refs):
    si = refs[0:R]
    di = refs[R:2 * R]
    rows = refs[2 * R:3 * R]
    ein = refs[3 * R:4 * R]
    agg_sh = refs[4 * R]
    sxs = refs[4 * R + 1:4 * R + 1 + R]
    sxd = refs[4 * R + 1 + R:4 * R + 1 + 2 * R]
    sg = refs[4 * R + 1 + 2 * R:4 * R + 1 + 3 * R]
    se = refs[4 * R + 1 + 3 * R:4 * R + 1 + 4 * R]
    ss = refs[4 * R + 1 + 4 * R:4 * R + 1 + 5 * R]
    c = lax.axis_index("c")
    s = lax.axis_index("s")
    wid = c * jnp.int32(NS) + s
    epw = E // NW
    nchunk = epw // C
    rpt = NP // NS
    kd = D // LANES

    def zro(r, _):
        for k in range(kd):
            rows[0][r, pl.ds(k * LANES, LANES)] = jnp.zeros((LANES,),
                                                            jnp.float32)
        return jnp.int32(0)
    _fori(C, zro)
    row0 = s * jnp.int32(rpt)
    nfull = rpt // C
    for t in range(nfull):
        pltpu.sync_copy(rows[0], agg_sh.at[pl.ds(row0 + t * C, C)])
    rem = rpt - nfull * C
    if rem:
        pltpu.sync_copy(rows[0].at[pl.ds(0, rem)],
                        agg_sh.at[pl.ds(row0 + nfull * C, rem)])
    plsc.subcore_barrier()

    ebase = wid * jnp.int32(epw)

    def issue_idx(j, z):
        pltpu.async_copy(src_hbm.at[pl.ds(ebase + j * jnp.int32(C), C)],
                         si[z], sxs[z])
        pltpu.async_copy(dst_hbm.at[pl.ds(ebase + j * jnp.int32(C), C)],
                         di[z], sxd[z])

    def issue_data(j, z):
        pltpu.make_async_copy(src_hbm.at[pl.ds(0, C)], si[z], sxs[z]).wait()
        pltpu.async_copy(x_hbm.at[si[z]], rows[z], sg[z])
        pltpu.async_copy(eattr_hbm.at[pl.ds(ebase + j * jnp.int32(C), C)],
                         ein[z], se[z])

    def process(j, z):
        pltpu.make_async_copy(eattr_hbm.at[pl.ds(0, C)], ein[z],
                              se[z]).wait()
        pltpu.make_async_copy(x_hbm.at[si[z]], rows[z], sg[z]).wait()

        def relu_row(r, _):
            for k in range(kd):
                sl = pl.ds(k * LANES, LANES)
                rows[z][r, sl] = jnp.maximum(
                    rows[z][r, sl] + ein[z][r, sl], 0.0)
            return jnp.int32(0)
        _fori(C, relu_row)

        pltpu.make_async_copy(dst_hbm.at[pl.ds(0, C)], di[z], sxd[z]).wait()
        dvec = di[z][...]
        pltpu.async_copy(rows[z], agg_sh.at[dvec], ss[z], add=True)

        @pl.when(j >= jnp.int32(R - LD))
        def _():
            pltpu.make_async_copy(rows[(z + LD) % R], agg_sh.at[dvec],
                                  ss[(z + LD) % R]).wait()

        @pl.when(j + jnp.int32(LI) < jnp.int32(nchunk))
        def _():
            issue_idx(j + jnp.int32(LI), (z + LI) % R)

        @pl.when(j + jnp.int32(LD) < jnp.int32(nchunk))
        def _():
            issue_data(j + jnp.int32(LD), (z + LD) % R)

    for q in range(LI):
        issue_idx(jnp.int32(q), q)
    for q in range(LD):
        issue_data(jnp.int32(q), q)

    def ring_loop(u, _):
        j0 = u * jnp.int32(R)
        for z in range(R):
            j = j0 + jnp.int32(z)

            @pl.when(j < jnp.int32(nchunk))
            def _(j=j, z=z):
                process(j, z)
        return jnp.int32(0)
    _fori((nchunk + R - 1) // R, ring_loop)

    dvec0 = di[0][...]
    for t in range(R - LD):
        pltpu.make_async_copy(rows[(nchunk - (R - LD) + t) % R],
                              agg_sh.at[dvec0],
                              ss[(nchunk - (R - LD) + t) % R]).wait()

    plsc.subcore_barrier()
    pltpu.sync_copy(agg_sh.at[pl.ds(row0, rpt)],
                    out_hbm.at[c, pl.ds(row0, rpt)])


def _sc_agg(x, eattr, src, dst, C=16, R=6, LI=6, LD=5):
    N, D = x.shape
    E = src.shape[0]
    rpt = (-(-N // NS) + 7) // 8 * 8
    NP = NS * rpt
    assert E % (NW * C) == 0 and D % LANES == 0
    mesh = plsc.VectorSubcoreMesh(core_axis_name="c", subcore_axis_name="s")
    f = pl.kernel(
        functools.partial(_sc_agg_body, NP, D, E, C, R, LI, LD),
        out_type=jax.ShapeDtypeStruct((NC, NP, D), jnp.float32),
        mesh=mesh,
        scratch_types=(
            [pltpu.VMEM((C,), jnp.int32)] * (2 * R)
            + [pltpu.VMEM((C, D), jnp.float32)] * (2 * R)
            + [pltpu.VMEM_SHARED((NP, D), jnp.float32)]
            + [pltpu.SemaphoreType.DMA] * (5 * R)
        ),
    )
    return f(x, eattr, src, dst)


def _mlp0_body(x_ref, p_ref, w1_ref, b1_ref, w2_ref, b2_ref, o_ref):
    n = x_ref.shape[0]
    out = x_ref[...] + p_ref[0][:n] + p_ref[1][:n]
    h = jnp.maximum(
        jnp.dot(out, w1_ref[...], preferred_element_type=jnp.float32)
        + b1_ref[...], 0.0)
    h = jnp.dot(h, w2_ref[...], preferred_element_type=jnp.float32) + b2_ref[...]
    o_ref[...] = jnp.maximum(h, 0.0)


def _final_body(x0_ref, h_ref, p_ref, w1_ref, b1_ref, w2_ref, b2_ref,
                lng_ref, lnb_ref, gnw_ref, gnb_ref, gna_ref, o_ref):
    n = h_ref.shape[0]
    out = h_ref[...] + p_ref[0][:n] + p_ref[1][:n]
    h = jnp.maximum(
        jnp.dot(out, w1_ref[...], preferred_element_type=jnp.float32)
        + b1_ref[...], 0.0)
    h = jnp.dot(h, w2_ref[...], preferred_element_type=jnp.float32) + b2_ref[...]
    mu = jnp.mean(h, axis=-1, keepdims=True)
    var = jnp.mean((h - mu) ** 2, axis=-1, keepdims=True)
    h = (h - mu) * lax.rsqrt(var + 1e-5) * lng_ref[...] + lnb_ref[...]
    gmean = jnp.mean(h, axis=0, keepdims=True)
    sub = h - gna_ref[...] * gmean
    gvar = jnp.mean(sub * sub, axis=0, keepdims=True)
    h = gnw_ref[...] * sub * lax.rsqrt(gvar + 1e-5) + gnb_ref[...]
    o_ref[...] = jnp.maximum(h, 0.0) + x0_ref[...]


def kernel(node_hidden, edge_hidden, edge_index, W1_0, b1_0, W2_0, b2_0,
           W1_1, b1_1, W2_1, b2_1, ln_gamma, ln_beta, gn_weight, gn_bias,
           gn_alpha):
    N, D = node_hidden.shape
    ei = edge_index.astype(jnp.int32)
    src, dst = ei[0], ei[1]
    x = node_hidden.astype(jnp.float32)
    eattr = edge_hidden.astype(jnp.float32)
    r2 = lambda v: v.reshape(1, D).astype(jnp.float32)

    p0 = _sc_agg(x, eattr, src, dst)
    h1 = pl.pallas_call(
        _mlp0_body,
        out_shape=jax.ShapeDtypeStruct((N, D), jnp.float32),
    )(x, p0, W1_0, r2(b1_0), W2_0, r2(b2_0))

    p1 = _sc_agg(h1, eattr, src, dst)
    out = pl.pallas_call(
        _final_body,
        out_shape=jax.ShapeDtypeStruct((N, D), jnp.float32),
    )(x, h1, p1, W1_1, r2(b1_1), W2_1, r2(b2_1),
      r2(ln_gamma), r2(ln_beta), r2(gn_weight), r2(gn_bias), r2(gn_alpha))
    return out
